# baseline (speedup 1.0000x reference)
import functools

import jax
import jax.numpy as jnp
from jax import lax
from jax.experimental import pallas as pl
from jax.experimental.pallas import tpu as pltpu
from jax.experimental.pallas import tpu_sc as plsc

VOC_NUM = 1000000
VOC_DIM = 64
BATCH = 16384
N_LOOKUPS = 4
CHUNK = 128
C = 256


def _make_sc_transpose():
    info = plsc.get_sparse_core_info()
    NC, NS = info.num_cores, info.num_subcores
    NW = NC * NS
    n_full = VOC_NUM // C
    rem = VOC_NUM - n_full * C
    k_per_w = n_full // NW
    n_extra = n_full - k_per_w * NW
    n_pairs = k_per_w // 2
    mesh = plsc.VectorSubcoreMesh(core_axis_name="c", subcore_axis_name="s")

    @functools.partial(
        pl.kernel,
        mesh=mesh,
        out_type=jax.ShapeDtypeStruct((VOC_NUM * VOC_DIM,), jnp.float32),
        scratch_types=[
            pltpu.VMEM((VOC_DIM, C), jnp.float32),
            pltpu.VMEM((VOC_DIM, C), jnp.float32),
            pltpu.VMEM((C * VOC_DIM,), jnp.float32),
            pltpu.VMEM((C * VOC_DIM,), jnp.float32),
            pltpu.VMEM((rem, VOC_DIM), jnp.float32),
            pltpu.SemaphoreType.DMA,
            pltpu.SemaphoreType.DMA,
            pltpu.SemaphoreType.DMA,
            pltpu.SemaphoreType.DMA,
        ],
        compiler_params=pltpu.CompilerParams(needs_layout_passes=False),
    )
    def transpose_k(tableT_hbm, rem_hbm, out_hbm, buf0_v, buf1_v,
                    pk0_v, pk1_v, rem_v, gi0, gi1, go0, go1):
        bufs = (buf0_v, buf1_v)
        pks = (pk0_v, pk1_v)
        wid = lax.axis_index("s") * NC + lax.axis_index("c")
        gis = (gi0, gi1)
        gos = (go0, go1)
        lane64 = lax.iota(jnp.int32, 16) * VOC_DIM

        def col_off(c):
            return pl.multiple_of(c * C, C)

        def out_off(c):
            return pl.multiple_of(c * (C * VOC_DIM), C * VOC_DIM)

        def fire_in(c, b):
            pltpu.async_copy(
                tableT_hbm.at[:, pl.ds(col_off(c), C)], bufs[b], gis[b])

        def wait_in(c, b):
            pltpu.make_async_copy(
                tableT_hbm.at[:, pl.ds(col_off(c), C)], bufs[b],
                gis[b]).wait()

        def fire_out(c, b):
            pltpu.async_copy(
                pks[b], out_hbm.at[pl.ds(out_off(c), C * VOC_DIM)],
                gos[b])

        def wait_out(c, b):
            pltpu.make_async_copy(
                pks[b], out_hbm.at[pl.ds(out_off(c), C * VOC_DIM)],
                gos[b]).wait()

        def transpose_chunk(b, n_cols=C):
            def d_body(d, carry):
                for g in range(n_cols // 16):
                    v = bufs[b][d, pl.ds(g * 16, 16)]
                    plsc.store_scatter(
                        pks[b], [lane64 + (g * 16 * VOC_DIM + d)], v)
                return carry
            lax.fori_loop(0, VOC_DIM, d_body, 0)

        fire_in(wid, 0)

        def pair_body(u, carry):
            c0 = wid + (2 * u) * NW
            c1 = c0 + NW
            c2 = c0 + 2 * NW
            wait_in(c0, 0)
            fire_in(c1, 1)
            pl.when(u > 0)(lambda: wait_out(c0 - 2 * NW, 0))
            transpose_chunk(0)
            fire_out(c0, 0)
            wait_in(c1, 1)
            pl.when(u + 1 < n_pairs)(lambda: fire_in(c2, 0))
            pl.when(u > 0)(lambda: wait_out(c1 - 2 * NW, 1))
            transpose_chunk(1)
            fire_out(c1, 1)
            return carry

        lax.fori_loop(0, n_pairs, pair_body, 0)
        wait_out(wid + (2 * n_pairs - 2) * NW, 0)
        wait_out(wid + (2 * n_pairs - 1) * NW, 1)

        @pl.when(wid < n_extra)
        def _():
            c = k_per_w * NW + wid
            pltpu.sync_copy(
                tableT_hbm.at[:, pl.ds(col_off(c), C)], buf0_v)
            transpose_chunk(0)
            pltpu.sync_copy(
                pk0_v, out_hbm.at[pl.ds(out_off(c), C * VOC_DIM)])

        if rem:
            @pl.when(wid == n_extra)
            def _():
                pltpu.sync_copy(rem_hbm, rem_v)
                for r in range(rem):
                    for q in range(VOC_DIM // 16):
                        pk0_v[pl.ds(r * VOC_DIM + q * 16, 16)] = (
                            rem_v[r, pl.ds(q * 16, 16)]
                        )
                pltpu.sync_copy(
                    pk0_v.at[pl.ds(0, rem * VOC_DIM)],
                    out_hbm.at[pl.ds(n_full * C * VOC_DIM,
                                     rem * VOC_DIM)])

    return transpose_k


def _make_sc_gather(B_total):
    info = plsc.get_sparse_core_info()
    NC, NS = info.num_cores, info.num_subcores
    NW = NC * NS
    b_per_w = B_total // NW
    n_chunks = b_per_w // CHUNK
    mesh = plsc.VectorSubcoreMesh(core_axis_name="c", subcore_axis_name="s")

    @functools.partial(
        pl.kernel,
        mesh=mesh,
        out_type=jax.ShapeDtypeStruct((B_total, VOC_DIM), jnp.float32),
        scratch_types=[
            pltpu.VMEM((b_per_w,), jnp.int32),
            pltpu.VMEM((2, CHUNK, VOC_DIM), jnp.float32),
            pltpu.SemaphoreType.DMA,
            pltpu.SemaphoreType.DMA,
        ],
        compiler_params=pltpu.CompilerParams(use_tc_tiling_on_sc=False),
    )
    def gather_k(idx_hbm, table_hbm, out_hbm, idx_v, rows_v, sem0, sem1):
        wid = lax.axis_index("s") * NC + lax.axis_index("c")
        base = wid * b_per_w
        pltpu.sync_copy(idx_hbm.at[pl.ds(base, b_per_w)], idx_v)
        sems = (sem0, sem1)
        copies = [None, None]
        for j in range(n_chunks + 1):
            if j < n_chunks:
                b = j & 1
                copies[b] = pltpu.async_copy(
                    table_hbm.at[idx_v.at[pl.ds(j * CHUNK, CHUNK)]],
                    rows_v.at[b],
                    sems[b],
                )
            if j >= 1:
                b2 = (j - 1) & 1
                copies[b2].wait()
                pltpu.sync_copy(
                    rows_v.at[b2],
                    out_hbm.at[pl.ds(base + (j - 1) * CHUNK, CHUNK)],
                )

    return gather_k


_sc_transpose = None
_sc_gather = None


def _get_kernels():
    global _sc_transpose, _sc_gather
    if _sc_transpose is None:
        _sc_transpose = _make_sc_transpose()
        _sc_gather = _make_sc_gather(N_LOOKUPS * BATCH)
    return _sc_transpose, _sc_gather


def _proj_body(g_ref, wt_ref, o_ref):
    acc = jnp.dot(g_ref[0], wt_ref[0], preferred_element_type=jnp.float32)
    for k in range(1, N_LOOKUPS):
        acc += jnp.dot(g_ref[k], wt_ref[k], preferred_element_type=jnp.float32)
    o_ref[...] = acc


def _tc_project(gathered, wt_stack):
    BB = 1024
    grid = (BATCH // BB,)
    return pl.pallas_call(
        _proj_body,
        grid=grid,
        in_specs=[
            pl.BlockSpec((N_LOOKUPS, BB, VOC_DIM), lambda i: (0, i, 0)),
            pl.BlockSpec((N_LOOKUPS, VOC_DIM, VOC_DIM), lambda i: (0, 0, 0)),
        ],
        out_specs=pl.BlockSpec((BB, VOC_DIM), lambda i: (i, 0)),
        out_shape=jax.ShapeDtypeStruct((BATCH, VOC_DIM), jnp.float32),
    )(gathered, wt_stack)


def kernel(x1, x2, x4, x5, codebook, W1, W2, W3, W4):
    transpose, gather = _get_kernels()
    idx_all = jnp.concatenate([x1, x2, x4, x5]).astype(jnp.int32)
    n_rem = VOC_NUM % C
    rem_block = lax.slice(codebook, (VOC_NUM - n_rem, 0),
                          (VOC_NUM, VOC_DIM))
    packed = transpose(codebook.T, rem_block)
    table_lin = packed.reshape(VOC_NUM, VOC_DIM)
    gathered = gather(idx_all, table_lin)
    gathered = gathered.reshape(N_LOOKUPS, BATCH, VOC_DIM)
    wt_stack = jnp.stack([W1.T, W2.T, W3.T, W4.T])
    return _tc_project(gathered, wt_stack)


# speedup vs baseline: 1.3590x; 1.3590x over previous
import functools

import jax
import jax.numpy as jnp
from jax import lax
from jax.experimental import pallas as pl
from jax.experimental.pallas import tpu as pltpu
from jax.experimental.pallas import tpu_sc as plsc

VOC_NUM = 1000000
VOC_DIM = 64
BATCH = 16384
N_LOOKUPS = 4
CHUNK = 128
C = 256


def _make_sc_transpose():
    info = plsc.get_sparse_core_info()
    NC, NS = info.num_cores, info.num_subcores
    NW = NC * NS
    n_full = VOC_NUM // C
    rem = VOC_NUM - n_full * C
    k_per_w = n_full // NW
    n_extra = n_full - k_per_w * NW
    n_pairs = k_per_w // 2
    mesh = plsc.VectorSubcoreMesh(core_axis_name="c", subcore_axis_name="s")

    @functools.partial(
        pl.kernel,
        mesh=mesh,
        out_type=jax.ShapeDtypeStruct((VOC_NUM * VOC_DIM,), jnp.float32),
        scratch_types=[
            pltpu.VMEM((VOC_DIM, C), jnp.float32),
            pltpu.VMEM((VOC_DIM, C), jnp.float32),
            pltpu.VMEM((C * VOC_DIM,), jnp.float32),
            pltpu.VMEM((C * VOC_DIM,), jnp.float32),
            pltpu.VMEM((rem, VOC_DIM), jnp.float32),
            pltpu.SemaphoreType.DMA,
            pltpu.SemaphoreType.DMA,
            pltpu.SemaphoreType.DMA,
            pltpu.SemaphoreType.DMA,
        ],
        compiler_params=pltpu.CompilerParams(needs_layout_passes=False),
    )
    def transpose_k(tableT_hbm, rem_hbm, out_hbm, buf0_v, buf1_v,
                    pk0_v, pk1_v, rem_v, gi0, gi1, go0, go1):
        bufs = (buf0_v, buf1_v)
        pks = (pk0_v, pk1_v)
        wid = lax.axis_index("s") * NC + lax.axis_index("c")
        gis = (gi0, gi1)
        gos = (go0, go1)
        lane64 = lax.iota(jnp.int32, 16) * VOC_DIM

        def col_off(c):
            return pl.multiple_of(c * C, C)

        def out_off(c):
            return pl.multiple_of(c * (C * VOC_DIM), C * VOC_DIM)

        def fire_in(c, b):
            pltpu.async_copy(
                tableT_hbm.at[:, pl.ds(col_off(c), C)], bufs[b], gis[b])

        def wait_in(c, b):
            pltpu.make_async_copy(
                tableT_hbm.at[:, pl.ds(col_off(c), C)], bufs[b],
                gis[b]).wait()

        def fire_out(c, b):
            pltpu.async_copy(
                pks[b], out_hbm.at[pl.ds(out_off(c), C * VOC_DIM)],
                gos[b])

        def wait_out(c, b):
            pltpu.make_async_copy(
                pks[b], out_hbm.at[pl.ds(out_off(c), C * VOC_DIM)],
                gos[b]).wait()

        def transpose_chunk(b, n_cols=C):
            @plsc.parallel_loop(0, VOC_DIM, unroll=4)
            def d_body(d):
                for g in range(n_cols // 16):
                    v = bufs[b][d, pl.ds(g * 16, 16)]
                    plsc.store_scatter(
                        pks[b], [lane64 + (g * 16 * VOC_DIM + d)], v)

        fire_in(wid, 0)

        def pair_body(u, carry):
            c0 = wid + (2 * u) * NW
            c1 = c0 + NW
            c2 = c0 + 2 * NW
            wait_in(c0, 0)
            fire_in(c1, 1)
            pl.when(u > 0)(lambda: wait_out(c0 - 2 * NW, 0))
            transpose_chunk(0)
            fire_out(c0, 0)
            wait_in(c1, 1)
            pl.when(u + 1 < n_pairs)(lambda: fire_in(c2, 0))
            pl.when(u > 0)(lambda: wait_out(c1 - 2 * NW, 1))
            transpose_chunk(1)
            fire_out(c1, 1)
            return carry

        lax.fori_loop(0, n_pairs, pair_body, 0)
        wait_out(wid + (2 * n_pairs - 2) * NW, 0)
        wait_out(wid + (2 * n_pairs - 1) * NW, 1)

        @pl.when(wid < n_extra)
        def _():
            c = k_per_w * NW + wid
            pltpu.sync_copy(
                tableT_hbm.at[:, pl.ds(col_off(c), C)], buf0_v)
            transpose_chunk(0)
            pltpu.sync_copy(
                pk0_v, out_hbm.at[pl.ds(out_off(c), C * VOC_DIM)])

        if rem:
            @pl.when(wid == n_extra)
            def _():
                pltpu.sync_copy(rem_hbm, rem_v)
                for r in range(rem):
                    for q in range(VOC_DIM // 16):
                        pk0_v[pl.ds(r * VOC_DIM + q * 16, 16)] = (
                            rem_v[r, pl.ds(q * 16, 16)]
                        )
                pltpu.sync_copy(
                    pk0_v.at[pl.ds(0, rem * VOC_DIM)],
                    out_hbm.at[pl.ds(n_full * C * VOC_DIM,
                                     rem * VOC_DIM)])

    return transpose_k


def _make_sc_gather(B_total):
    info = plsc.get_sparse_core_info()
    NC, NS = info.num_cores, info.num_subcores
    NW = NC * NS
    b_per_w = B_total // NW
    n_chunks = b_per_w // CHUNK
    mesh = plsc.VectorSubcoreMesh(core_axis_name="c", subcore_axis_name="s")

    @functools.partial(
        pl.kernel,
        mesh=mesh,
        out_type=jax.ShapeDtypeStruct((B_total, VOC_DIM), jnp.float32),
        scratch_types=[
            pltpu.VMEM((b_per_w,), jnp.int32),
            pltpu.VMEM((2, CHUNK, VOC_DIM), jnp.float32),
            pltpu.SemaphoreType.DMA,
            pltpu.SemaphoreType.DMA,
        ],
        compiler_params=pltpu.CompilerParams(use_tc_tiling_on_sc=False),
    )
    def gather_k(idx_hbm, table_hbm, out_hbm, idx_v, rows_v, sem0, sem1):
        wid = lax.axis_index("s") * NC + lax.axis_index("c")
        base = wid * b_per_w
        pltpu.sync_copy(idx_hbm.at[pl.ds(base, b_per_w)], idx_v)
        sems = (sem0, sem1)
        copies = [None, None]
        for j in range(n_chunks + 1):
            if j < n_chunks:
                b = j & 1
                copies[b] = pltpu.async_copy(
                    table_hbm.at[idx_v.at[pl.ds(j * CHUNK, CHUNK)]],
                    rows_v.at[b],
                    sems[b],
                )
            if j >= 1:
                b2 = (j - 1) & 1
                copies[b2].wait()
                pltpu.sync_copy(
                    rows_v.at[b2],
                    out_hbm.at[pl.ds(base + (j - 1) * CHUNK, CHUNK)],
                )

    return gather_k


_sc_transpose = None
_sc_gather = None


def _get_kernels():
    global _sc_transpose, _sc_gather
    if _sc_transpose is None:
        _sc_transpose = _make_sc_transpose()
        _sc_gather = _make_sc_gather(N_LOOKUPS * BATCH)
    return _sc_transpose, _sc_gather


def _proj_body(g_ref, wt_ref, o_ref):
    acc = jnp.dot(g_ref[0], wt_ref[0], preferred_element_type=jnp.float32)
    for k in range(1, N_LOOKUPS):
        acc += jnp.dot(g_ref[k], wt_ref[k], preferred_element_type=jnp.float32)
    o_ref[...] = acc


def _tc_project(gathered, wt_stack):
    BB = 1024
    grid = (BATCH // BB,)
    return pl.pallas_call(
        _proj_body,
        grid=grid,
        in_specs=[
            pl.BlockSpec((N_LOOKUPS, BB, VOC_DIM), lambda i: (0, i, 0)),
            pl.BlockSpec((N_LOOKUPS, VOC_DIM, VOC_DIM), lambda i: (0, 0, 0)),
        ],
        out_specs=pl.BlockSpec((BB, VOC_DIM), lambda i: (i, 0)),
        out_shape=jax.ShapeDtypeStruct((BATCH, VOC_DIM), jnp.float32),
    )(gathered, wt_stack)


def kernel(x1, x2, x4, x5, codebook, W1, W2, W3, W4):
    transpose, gather = _get_kernels()
    idx_all = jnp.concatenate([x1, x2, x4, x5]).astype(jnp.int32)
    n_rem = VOC_NUM % C
    rem_block = lax.slice(codebook, (VOC_NUM - n_rem, 0),
                          (VOC_NUM, VOC_DIM))
    packed = transpose(codebook.T, rem_block)
    table_lin = packed.reshape(VOC_NUM, VOC_DIM)
    gathered = gather(idx_all, table_lin)
    gathered = gathered.reshape(N_LOOKUPS, BATCH, VOC_DIM)
    wt_stack = jnp.stack([W1.T, W2.T, W3.T, W4.T])
    return _tc_project(gathered, wt_stack)


# speedup vs baseline: 2.0027x; 1.4736x over previous
import functools

import jax
import jax.numpy as jnp
from jax import lax
from jax.experimental import pallas as pl
from jax.experimental.pallas import tpu as pltpu
from jax.experimental.pallas import tpu_sc as plsc

VOC_NUM = 1000000
VOC_DIM = 64
BATCH = 16384
N_LOOKUPS = 4
CHUNK = 128
C = 256


def _make_sc_transpose():
    info = plsc.get_sparse_core_info()
    NC, NS = info.num_cores, info.num_subcores
    NW = NC * NS
    n_full = VOC_NUM // C
    rem = VOC_NUM - n_full * C
    k_per_w = n_full // NW
    n_extra = n_full - k_per_w * NW
    n_pairs = k_per_w // 2
    mesh = plsc.VectorSubcoreMesh(core_axis_name="c", subcore_axis_name="s")

    @functools.partial(
        pl.kernel,
        mesh=mesh,
        out_type=jax.ShapeDtypeStruct((VOC_NUM * VOC_DIM,), jnp.float32),
        scratch_types=[
            pltpu.VMEM((VOC_DIM, C), jnp.float32),
            pltpu.VMEM((VOC_DIM, C), jnp.float32),
            pltpu.VMEM((C * VOC_DIM,), jnp.float32),
            pltpu.VMEM((C * VOC_DIM,), jnp.float32),
            pltpu.VMEM((rem, VOC_DIM), jnp.float32),
            pltpu.SemaphoreType.DMA,
            pltpu.SemaphoreType.DMA,
            pltpu.SemaphoreType.DMA,
            pltpu.SemaphoreType.DMA,
        ],
        compiler_params=pltpu.CompilerParams(needs_layout_passes=False),
    )
    def transpose_k(tableT_hbm, rem_hbm, out_hbm, buf0_v, buf1_v,
                    pk0_v, pk1_v, rem_v, gi0, gi1, go0, go1):
        bufs = (buf0_v, buf1_v)
        pks = (pk0_v, pk1_v)
        wid = lax.axis_index("s") * NC + lax.axis_index("c")
        gis = (gi0, gi1)
        gos = (go0, go1)
        lane = lax.iota(jnp.int32, 16)
        lane64 = lane * VOC_DIM
        diags = [(lane + s) & 15 for s in range(16)]

        def col_off(c):
            return pl.multiple_of(c * C, C)

        def out_off(c):
            return pl.multiple_of(c * (C * VOC_DIM), C * VOC_DIM)

        def fire_in(c, b):
            pltpu.async_copy(
                tableT_hbm.at[:, pl.ds(col_off(c), C)], bufs[b], gis[b])

        def wait_in(c, b):
            pltpu.make_async_copy(
                tableT_hbm.at[:, pl.ds(col_off(c), C)], bufs[b],
                gis[b]).wait()

        def fire_out(c, b):
            pltpu.async_copy(
                pks[b], out_hbm.at[pl.ds(out_off(c), C * VOC_DIM)],
                gos[b])

        def wait_out(c, b):
            pltpu.make_async_copy(
                pks[b], out_hbm.at[pl.ds(out_off(c), C * VOC_DIM)],
                gos[b]).wait()

        def transpose_chunk(b, n_cols=C):
            @plsc.parallel_loop(0, VOC_DIM // 16)
            def d0_body(d0):
                d_base = d0 * 16
                for s in range(16):
                    dvec = diags[s] + d_base
                    sbase = lane64 + dvec
                    for j0 in range(n_cols // 16):
                        v = plsc.load_gather(
                            bufs[b], [dvec, lane + (j0 * 16)])
                        plsc.store_scatter(
                            pks[b], [sbase + (j0 * 16 * VOC_DIM)], v)

        fire_in(wid, 0)

        def pair_body(u, carry):
            c0 = wid + (2 * u) * NW
            c1 = c0 + NW
            c2 = c0 + 2 * NW
            wait_in(c0, 0)
            fire_in(c1, 1)
            pl.when(u > 0)(lambda: wait_out(c0 - 2 * NW, 0))
            transpose_chunk(0)
            fire_out(c0, 0)
            wait_in(c1, 1)
            pl.when(u + 1 < n_pairs)(lambda: fire_in(c2, 0))
            pl.when(u > 0)(lambda: wait_out(c1 - 2 * NW, 1))
            transpose_chunk(1)
            fire_out(c1, 1)
            return carry

        lax.fori_loop(0, n_pairs, pair_body, 0)
        wait_out(wid + (2 * n_pairs - 2) * NW, 0)
        wait_out(wid + (2 * n_pairs - 1) * NW, 1)

        @pl.when(wid < n_extra)
        def _():
            c = k_per_w * NW + wid
            pltpu.sync_copy(
                tableT_hbm.at[:, pl.ds(col_off(c), C)], buf0_v)
            transpose_chunk(0)
            pltpu.sync_copy(
                pk0_v, out_hbm.at[pl.ds(out_off(c), C * VOC_DIM)])

        if rem:
            @pl.when(wid == n_extra)
            def _():
                pltpu.sync_copy(rem_hbm, rem_v)
                for r in range(rem):
                    for q in range(VOC_DIM // 16):
                        pk0_v[pl.ds(r * VOC_DIM + q * 16, 16)] = (
                            rem_v[r, pl.ds(q * 16, 16)]
                        )
                pltpu.sync_copy(
                    pk0_v.at[pl.ds(0, rem * VOC_DIM)],
                    out_hbm.at[pl.ds(n_full * C * VOC_DIM,
                                     rem * VOC_DIM)])

    return transpose_k


def _make_sc_gather(B_total):
    info = plsc.get_sparse_core_info()
    NC, NS = info.num_cores, info.num_subcores
    NW = NC * NS
    b_per_w = B_total // NW
    n_chunks = b_per_w // CHUNK
    mesh = plsc.VectorSubcoreMesh(core_axis_name="c", subcore_axis_name="s")

    @functools.partial(
        pl.kernel,
        mesh=mesh,
        out_type=jax.ShapeDtypeStruct((B_total, VOC_DIM), jnp.float32),
        scratch_types=[
            pltpu.VMEM((b_per_w,), jnp.int32),
            pltpu.VMEM((2, CHUNK, VOC_DIM), jnp.float32),
            pltpu.SemaphoreType.DMA,
            pltpu.SemaphoreType.DMA,
        ],
        compiler_params=pltpu.CompilerParams(use_tc_tiling_on_sc=False),
    )
    def gather_k(idx_hbm, table_hbm, out_hbm, idx_v, rows_v, sem0, sem1):
        wid = lax.axis_index("s") * NC + lax.axis_index("c")
        base = wid * b_per_w
        pltpu.sync_copy(idx_hbm.at[pl.ds(base, b_per_w)], idx_v)
        sems = (sem0, sem1)
        copies = [None, None]
        for j in range(n_chunks + 1):
            if j < n_chunks:
                b = j & 1
                copies[b] = pltpu.async_copy(
                    table_hbm.at[idx_v.at[pl.ds(j * CHUNK, CHUNK)]],
                    rows_v.at[b],
                    sems[b],
                )
            if j >= 1:
                b2 = (j - 1) & 1
                copies[b2].wait()
                pltpu.sync_copy(
                    rows_v.at[b2],
                    out_hbm.at[pl.ds(base + (j - 1) * CHUNK, CHUNK)],
                )

    return gather_k


_sc_transpose = None
_sc_gather = None


def _get_kernels():
    global _sc_transpose, _sc_gather
    if _sc_transpose is None:
        _sc_transpose = _make_sc_transpose()
        _sc_gather = _make_sc_gather(N_LOOKUPS * BATCH)
    return _sc_transpose, _sc_gather


def _proj_body(g_ref, wt_ref, o_ref):
    acc = jnp.dot(g_ref[0], wt_ref[0], preferred_element_type=jnp.float32)
    for k in range(1, N_LOOKUPS):
        acc += jnp.dot(g_ref[k], wt_ref[k], preferred_element_type=jnp.float32)
    o_ref[...] = acc


def _tc_project(gathered, wt_stack):
    BB = 1024
    grid = (BATCH // BB,)
    return pl.pallas_call(
        _proj_body,
        grid=grid,
        in_specs=[
            pl.BlockSpec((N_LOOKUPS, BB, VOC_DIM), lambda i: (0, i, 0)),
            pl.BlockSpec((N_LOOKUPS, VOC_DIM, VOC_DIM), lambda i: (0, 0, 0)),
        ],
        out_specs=pl.BlockSpec((BB, VOC_DIM), lambda i: (i, 0)),
        out_shape=jax.ShapeDtypeStruct((BATCH, VOC_DIM), jnp.float32),
    )(gathered, wt_stack)


def kernel(x1, x2, x4, x5, codebook, W1, W2, W3, W4):
    transpose, gather = _get_kernels()
    idx_all = jnp.concatenate([x1, x2, x4, x5]).astype(jnp.int32)
    n_rem = VOC_NUM % C
    rem_block = lax.slice(codebook, (VOC_NUM - n_rem, 0),
                          (VOC_NUM, VOC_DIM))
    packed = transpose(codebook.T, rem_block)
    table_lin = packed.reshape(VOC_NUM, VOC_DIM)
    gathered = gather(idx_all, table_lin)
    gathered = gathered.reshape(N_LOOKUPS, BATCH, VOC_DIM)
    wt_stack = jnp.stack([W1.T, W2.T, W3.T, W4.T])
    return _tc_project(gathered, wt_stack)


# speedup vs baseline: 3.1516x; 1.5737x over previous
import functools

import jax
import jax.numpy as jnp
from jax import lax
from jax.experimental import pallas as pl
from jax.experimental.pallas import tpu as pltpu
from jax.experimental.pallas import tpu_sc as plsc

VOC_NUM = 1000000
VOC_DIM = 64
BATCH = 16384
N_LOOKUPS = 4
CHUNK = 128
C = 256


def _make_sc_transpose():
    info = plsc.get_sparse_core_info()
    NC, NS = info.num_cores, info.num_subcores
    NW = NC * NS
    n_full = VOC_NUM // C
    rem = VOC_NUM - n_full * C
    k_per_w = n_full // NW
    n_extra = n_full - k_per_w * NW
    n_pairs = k_per_w // 2
    mesh = plsc.VectorSubcoreMesh(core_axis_name="c", subcore_axis_name="s")

    @functools.partial(
        pl.kernel,
        mesh=mesh,
        out_type=jax.ShapeDtypeStruct((VOC_NUM * VOC_DIM,), jnp.float32),
        scratch_types=[
            pltpu.VMEM((VOC_DIM, C), jnp.float32),
            pltpu.VMEM((VOC_DIM, C), jnp.float32),
            pltpu.VMEM((C * VOC_DIM,), jnp.float32),
            pltpu.VMEM((C * VOC_DIM,), jnp.float32),
            pltpu.VMEM((rem, VOC_DIM), jnp.float32),
            pltpu.VMEM((16 * 16,), jnp.int32),
            pltpu.VMEM((16 * 16,), jnp.int32),
            pltpu.VMEM((16,), jnp.int32),
            pltpu.SemaphoreType.DMA,
            pltpu.SemaphoreType.DMA,
            pltpu.SemaphoreType.DMA,
            pltpu.SemaphoreType.DMA,
        ],
        compiler_params=pltpu.CompilerParams(needs_layout_passes=False),
    )
    def transpose_k(tableT_hbm, rem_hbm, out_hbm, buf0_v, buf1_v,
                    pk0_v, pk1_v, rem_v, gd_v, sd_v, lane_v,
                    gi0, gi1, go0, go1):
        bufs = (buf0_v, buf1_v)
        pks = (pk0_v, pk1_v)
        wid = lax.axis_index("s") * NC + lax.axis_index("c")
        gis = (gi0, gi1)
        gos = (go0, go1)
        lane = lax.iota(jnp.int32, 16)
        lane64 = lane * VOC_DIM
        for s in range(16):
            diag = (lane + s) & 15
            gd_v[pl.ds(s * 16, 16)] = diag
            sd_v[pl.ds(s * 16, 16)] = lane64 + diag
        lane_v[pl.ds(0, 16)] = lane

        def col_off(c):
            return pl.multiple_of(c * C, C)

        def out_off(c):
            return pl.multiple_of(c * (C * VOC_DIM), C * VOC_DIM)

        def fire_in(c, b):
            pltpu.async_copy(
                tableT_hbm.at[:, pl.ds(col_off(c), C)], bufs[b], gis[b])

        def wait_in(c, b):
            pltpu.make_async_copy(
                tableT_hbm.at[:, pl.ds(col_off(c), C)], bufs[b],
                gis[b]).wait()

        def fire_out(c, b):
            pltpu.async_copy(
                pks[b], out_hbm.at[pl.ds(out_off(c), C * VOC_DIM)],
                gos[b])

        def wait_out(c, b):
            pltpu.make_async_copy(
                pks[b], out_hbm.at[pl.ds(out_off(c), C * VOC_DIM)],
                gos[b]).wait()

        def transpose_chunk(b, n_cols=C):
            @plsc.parallel_loop(0, 16)
            def s_body(s):
                off = pl.multiple_of(s * 16, 16)
                gd = gd_v[pl.ds(off, 16)]
                sd = sd_v[pl.ds(off, 16)]
                lv = lane_v[pl.ds(0, 16)]

                @plsc.parallel_loop(0, VOC_DIM // 16)
                def d0_body(d0, gd=gd, sd=sd, lv=lv):
                    d_base = d0 * 16
                    dvec = gd + d_base
                    jv = lv
                    for j0 in range(n_cols // 16):
                        v = plsc.load_gather(bufs[b], [dvec, jv])
                        plsc.store_scatter(
                            pks[b],
                            [sd + (d_base + j0 * 16 * VOC_DIM)], v)
                        if j0 + 1 < n_cols // 16:
                            jv = jv + 16

        fire_in(wid, 0)

        def pair_body(u, carry):
            c0 = wid + (2 * u) * NW
            c1 = c0 + NW
            c2 = c0 + 2 * NW
            wait_in(c0, 0)
            fire_in(c1, 1)
            pl.when(u > 0)(lambda: wait_out(c0 - 2 * NW, 0))
            transpose_chunk(0)
            fire_out(c0, 0)
            wait_in(c1, 1)
            pl.when(u + 1 < n_pairs)(lambda: fire_in(c2, 0))
            pl.when(u > 0)(lambda: wait_out(c1 - 2 * NW, 1))
            transpose_chunk(1)
            fire_out(c1, 1)
            return carry

        lax.fori_loop(0, n_pairs, pair_body, 0)
        wait_out(wid + (2 * n_pairs - 2) * NW, 0)
        wait_out(wid + (2 * n_pairs - 1) * NW, 1)

        @pl.when(wid < n_extra)
        def _():
            c = k_per_w * NW + wid
            pltpu.sync_copy(
                tableT_hbm.at[:, pl.ds(col_off(c), C)], buf0_v)
            transpose_chunk(0)
            pltpu.sync_copy(
                pk0_v, out_hbm.at[pl.ds(out_off(c), C * VOC_DIM)])

        if rem:
            @pl.when(wid == n_extra)
            def _():
                pltpu.sync_copy(rem_hbm, rem_v)
                for r in range(rem):
                    for q in range(VOC_DIM // 16):
                        pk0_v[pl.ds(r * VOC_DIM + q * 16, 16)] = (
                            rem_v[r, pl.ds(q * 16, 16)]
                        )
                pltpu.sync_copy(
                    pk0_v.at[pl.ds(0, rem * VOC_DIM)],
                    out_hbm.at[pl.ds(n_full * C * VOC_DIM,
                                     rem * VOC_DIM)])

    return transpose_k


def _make_sc_gather(B_total):
    info = plsc.get_sparse_core_info()
    NC, NS = info.num_cores, info.num_subcores
    NW = NC * NS
    b_per_w = B_total // NW
    n_chunks = b_per_w // CHUNK
    mesh = plsc.VectorSubcoreMesh(core_axis_name="c", subcore_axis_name="s")

    @functools.partial(
        pl.kernel,
        mesh=mesh,
        out_type=jax.ShapeDtypeStruct((B_total, VOC_DIM), jnp.float32),
        scratch_types=[
            pltpu.VMEM((b_per_w,), jnp.int32),
            pltpu.VMEM((2, CHUNK, VOC_DIM), jnp.float32),
            pltpu.SemaphoreType.DMA,
            pltpu.SemaphoreType.DMA,
        ],
        compiler_params=pltpu.CompilerParams(use_tc_tiling_on_sc=False),
    )
    def gather_k(idx_hbm, table_hbm, out_hbm, idx_v, rows_v, sem0, sem1):
        wid = lax.axis_index("s") * NC + lax.axis_index("c")
        base = wid * b_per_w
        pltpu.sync_copy(idx_hbm.at[pl.ds(base, b_per_w)], idx_v)
        sems = (sem0, sem1)
        copies = [None, None]
        for j in range(n_chunks + 1):
            if j < n_chunks:
                b = j & 1
                copies[b] = pltpu.async_copy(
                    table_hbm.at[idx_v.at[pl.ds(j * CHUNK, CHUNK)]],
                    rows_v.at[b],
                    sems[b],
                )
            if j >= 1:
                b2 = (j - 1) & 1
                copies[b2].wait()
                pltpu.sync_copy(
                    rows_v.at[b2],
                    out_hbm.at[pl.ds(base + (j - 1) * CHUNK, CHUNK)],
                )

    return gather_k


_sc_transpose = None
_sc_gather = None


def _get_kernels():
    global _sc_transpose, _sc_gather
    if _sc_transpose is None:
        _sc_transpose = _make_sc_transpose()
        _sc_gather = _make_sc_gather(N_LOOKUPS * BATCH)
    return _sc_transpose, _sc_gather


def _proj_body(g_ref, wt_ref, o_ref):
    acc = jnp.dot(g_ref[0], wt_ref[0], preferred_element_type=jnp.float32)
    for k in range(1, N_LOOKUPS):
        acc += jnp.dot(g_ref[k], wt_ref[k], preferred_element_type=jnp.float32)
    o_ref[...] = acc


def _tc_project(gathered, wt_stack):
    BB = 1024
    grid = (BATCH // BB,)
    return pl.pallas_call(
        _proj_body,
        grid=grid,
        in_specs=[
            pl.BlockSpec((N_LOOKUPS, BB, VOC_DIM), lambda i: (0, i, 0)),
            pl.BlockSpec((N_LOOKUPS, VOC_DIM, VOC_DIM), lambda i: (0, 0, 0)),
        ],
        out_specs=pl.BlockSpec((BB, VOC_DIM), lambda i: (i, 0)),
        out_shape=jax.ShapeDtypeStruct((BATCH, VOC_DIM), jnp.float32),
    )(gathered, wt_stack)


def kernel(x1, x2, x4, x5, codebook, W1, W2, W3, W4):
    transpose, gather = _get_kernels()
    idx_all = jnp.concatenate([x1, x2, x4, x5]).astype(jnp.int32)
    n_rem = VOC_NUM % C
    rem_block = lax.slice(codebook, (VOC_NUM - n_rem, 0),
                          (VOC_NUM, VOC_DIM))
    packed = transpose(codebook.T, rem_block)
    table_lin = packed.reshape(VOC_NUM, VOC_DIM)
    gathered = gather(idx_all, table_lin)
    gathered = gathered.reshape(N_LOOKUPS, BATCH, VOC_DIM)
    wt_stack = jnp.stack([W1.T, W2.T, W3.T, W4.T])
    return _tc_project(gathered, wt_stack)


# speedup vs baseline: 3.3769x; 1.0715x over previous
import functools

import jax
import jax.numpy as jnp
from jax import lax
from jax.experimental import pallas as pl
from jax.experimental.pallas import tpu as pltpu
from jax.experimental.pallas import tpu_sc as plsc

VOC_NUM = 1000000
VOC_DIM = 64
BATCH = 16384
N_LOOKUPS = 4
CHUNK = 128
C = 128


def _make_sc_transpose():
    info = plsc.get_sparse_core_info()
    NC, NS = info.num_cores, info.num_subcores
    NW = NC * NS
    n_full = VOC_NUM // C
    rem = VOC_NUM - n_full * C
    k_per_w = n_full // NW
    n_extra = n_full - k_per_w * NW
    n_pairs = k_per_w // 2
    mesh = plsc.VectorSubcoreMesh(core_axis_name="c", subcore_axis_name="s")

    @functools.partial(
        pl.kernel,
        mesh=mesh,
        out_type=jax.ShapeDtypeStruct((VOC_NUM * VOC_DIM,), jnp.float32),
        scratch_types=[
            pltpu.VMEM((VOC_DIM, C), jnp.float32),
            pltpu.VMEM((VOC_DIM, C), jnp.float32),
            pltpu.VMEM((C * VOC_DIM,), jnp.float32),
            pltpu.VMEM((C * VOC_DIM,), jnp.float32),
            pltpu.VMEM((rem, VOC_DIM), jnp.float32),
            pltpu.VMEM((16 * 16,), jnp.int32),
            pltpu.VMEM((16 * 16,), jnp.int32),
            pltpu.VMEM((16,), jnp.int32),
            pltpu.SemaphoreType.DMA,
            pltpu.SemaphoreType.DMA,
            pltpu.SemaphoreType.DMA,
            pltpu.SemaphoreType.DMA,
        ],
        compiler_params=pltpu.CompilerParams(needs_layout_passes=False),
    )
    def transpose_k(tableT_hbm, rem_hbm, out_hbm, buf0_v, buf1_v,
                    pk0_v, pk1_v, rem_v, gd_v, sd_v, lane_v,
                    gi0, gi1, go0, go1):
        bufs = (buf0_v, buf1_v)
        pks = (pk0_v, pk1_v)
        wid = lax.axis_index("s") * NC + lax.axis_index("c")
        gis = (gi0, gi1)
        gos = (go0, go1)
        lane = lax.iota(jnp.int32, 16)
        lane64 = lane * VOC_DIM
        for s in range(16):
            diag = (lane + s) & 15
            gd_v[pl.ds(s * 16, 16)] = diag
            sd_v[pl.ds(s * 16, 16)] = lane64 + diag
        lane_v[pl.ds(0, 16)] = lane

        def col_off(c):
            return pl.multiple_of(c * C, C)

        def out_off(c):
            return pl.multiple_of(c * (C * VOC_DIM), C * VOC_DIM)

        def fire_in(c, b):
            pltpu.async_copy(
                tableT_hbm.at[:, pl.ds(col_off(c), C)], bufs[b], gis[b])

        def wait_in(c, b):
            pltpu.make_async_copy(
                tableT_hbm.at[:, pl.ds(col_off(c), C)], bufs[b],
                gis[b]).wait()

        def fire_out(c, b):
            pltpu.async_copy(
                pks[b], out_hbm.at[pl.ds(out_off(c), C * VOC_DIM)],
                gos[b])

        def wait_out(c, b):
            pltpu.make_async_copy(
                pks[b], out_hbm.at[pl.ds(out_off(c), C * VOC_DIM)],
                gos[b]).wait()

        def transpose_chunk(b, n_cols=C):
            @plsc.parallel_loop(0, 16 * (VOC_DIM // 16), unroll=2)
            def sd_body(i):
                s = lax.shift_right_logical(i, 2)
                d0 = lax.bitwise_and(i, 3)
                off = pl.multiple_of(s * 16, 16)
                gd = gd_v[pl.ds(off, 16)]
                sd = sd_v[pl.ds(off, 16)]
                lv = lane_v[pl.ds(0, 16)]
                d_base = d0 * 16
                dvec = gd + d_base
                sbase = sd + d_base
                jv = lv
                for j0 in range(n_cols // 16):
                    v = plsc.load_gather(bufs[b], [dvec, jv])
                    plsc.store_scatter(
                        pks[b], [sbase + (j0 * 16 * VOC_DIM)], v)
                    if j0 + 1 < n_cols // 16:
                        jv = jv + 16

        fire_in(wid, 0)

        def pair_body(u, carry):
            c0 = wid + (2 * u) * NW
            c1 = c0 + NW
            c2 = c0 + 2 * NW
            wait_in(c0, 0)
            fire_in(c1, 1)
            pl.when(u > 0)(lambda: wait_out(c0 - 2 * NW, 0))
            transpose_chunk(0)
            fire_out(c0, 0)
            wait_in(c1, 1)
            pl.when(u + 1 < n_pairs)(lambda: fire_in(c2, 0))
            pl.when(u > 0)(lambda: wait_out(c1 - 2 * NW, 1))
            transpose_chunk(1)
            fire_out(c1, 1)
            return carry

        lax.fori_loop(0, n_pairs, pair_body, 0)
        wait_out(wid + (2 * n_pairs - 2) * NW, 0)
        wait_out(wid + (2 * n_pairs - 1) * NW, 1)

        @pl.when(wid < n_extra)
        def _():
            c = k_per_w * NW + wid
            pltpu.sync_copy(
                tableT_hbm.at[:, pl.ds(col_off(c), C)], buf0_v)
            transpose_chunk(0)
            pltpu.sync_copy(
                pk0_v, out_hbm.at[pl.ds(out_off(c), C * VOC_DIM)])

        if rem:
            @pl.when(wid == n_extra)
            def _():
                pltpu.sync_copy(rem_hbm, rem_v)
                for r in range(rem):
                    for q in range(VOC_DIM // 16):
                        pk0_v[pl.ds(r * VOC_DIM + q * 16, 16)] = (
                            rem_v[r, pl.ds(q * 16, 16)]
                        )
                pltpu.sync_copy(
                    pk0_v.at[pl.ds(0, rem * VOC_DIM)],
                    out_hbm.at[pl.ds(n_full * C * VOC_DIM,
                                     rem * VOC_DIM)])

    return transpose_k


def _make_sc_gather(B_total):
    info = plsc.get_sparse_core_info()
    NC, NS = info.num_cores, info.num_subcores
    NW = NC * NS
    b_per_w = B_total // NW
    n_chunks = b_per_w // CHUNK
    mesh = plsc.VectorSubcoreMesh(core_axis_name="c", subcore_axis_name="s")

    @functools.partial(
        pl.kernel,
        mesh=mesh,
        out_type=jax.ShapeDtypeStruct((B_total, VOC_DIM), jnp.float32),
        scratch_types=[
            pltpu.VMEM((b_per_w,), jnp.int32),
            pltpu.VMEM((2, CHUNK, VOC_DIM), jnp.float32),
            pltpu.SemaphoreType.DMA,
            pltpu.SemaphoreType.DMA,
        ],
        compiler_params=pltpu.CompilerParams(use_tc_tiling_on_sc=False),
    )
    def gather_k(idx_hbm, table_hbm, out_hbm, idx_v, rows_v, sem0, sem1):
        wid = lax.axis_index("s") * NC + lax.axis_index("c")
        base = wid * b_per_w
        pltpu.sync_copy(idx_hbm.at[pl.ds(base, b_per_w)], idx_v)
        sems = (sem0, sem1)
        copies = [None, None]
        for j in range(n_chunks + 1):
            if j < n_chunks:
                b = j & 1
                copies[b] = pltpu.async_copy(
                    table_hbm.at[idx_v.at[pl.ds(j * CHUNK, CHUNK)]],
                    rows_v.at[b],
                    sems[b],
                )
            if j >= 1:
                b2 = (j - 1) & 1
                copies[b2].wait()
                pltpu.sync_copy(
                    rows_v.at[b2],
                    out_hbm.at[pl.ds(base + (j - 1) * CHUNK, CHUNK)],
                )

    return gather_k


_sc_transpose = None
_sc_gather = None


def _get_kernels():
    global _sc_transpose, _sc_gather
    if _sc_transpose is None:
        _sc_transpose = _make_sc_transpose()
        _sc_gather = _make_sc_gather(N_LOOKUPS * BATCH)
    return _sc_transpose, _sc_gather


def _proj_body(g_ref, wt_ref, o_ref):
    acc = jnp.dot(g_ref[0], wt_ref[0], preferred_element_type=jnp.float32)
    for k in range(1, N_LOOKUPS):
        acc += jnp.dot(g_ref[k], wt_ref[k], preferred_element_type=jnp.float32)
    o_ref[...] = acc


def _tc_project(gathered, wt_stack):
    BB = 1024
    grid = (BATCH // BB,)
    return pl.pallas_call(
        _proj_body,
        grid=grid,
        in_specs=[
            pl.BlockSpec((N_LOOKUPS, BB, VOC_DIM), lambda i: (0, i, 0)),
            pl.BlockSpec((N_LOOKUPS, VOC_DIM, VOC_DIM), lambda i: (0, 0, 0)),
        ],
        out_specs=pl.BlockSpec((BB, VOC_DIM), lambda i: (i, 0)),
        out_shape=jax.ShapeDtypeStruct((BATCH, VOC_DIM), jnp.float32),
    )(gathered, wt_stack)


def kernel(x1, x2, x4, x5, codebook, W1, W2, W3, W4):
    transpose, gather = _get_kernels()
    idx_all = jnp.concatenate([x1, x2, x4, x5]).astype(jnp.int32)
    n_rem = VOC_NUM % C
    rem_block = lax.slice(codebook, (VOC_NUM - n_rem, 0),
                          (VOC_NUM, VOC_DIM))
    packed = transpose(codebook.T, rem_block)
    table_lin = packed.reshape(VOC_NUM, VOC_DIM)
    gathered = gather(idx_all, table_lin)
    gathered = gathered.reshape(N_LOOKUPS, BATCH, VOC_DIM)
    wt_stack = jnp.stack([W1.T, W2.T, W3.T, W4.T])
    return _tc_project(gathered, wt_stack)


# speedup vs baseline: 3.5931x; 1.0640x over previous
import functools

import jax
import jax.numpy as jnp
from jax import lax
from jax.experimental import pallas as pl
from jax.experimental.pallas import tpu as pltpu
from jax.experimental.pallas import tpu_sc as plsc

VOC_NUM = 1000000
VOC_DIM = 64
BATCH = 16384
N_LOOKUPS = 4
CHUNK = 128
C = 128


def _make_sc_transpose():
    info = plsc.get_sparse_core_info()
    NC, NS = info.num_cores, info.num_subcores
    NW = NC * NS
    n_full = VOC_NUM // C
    rem = VOC_NUM - n_full * C
    k_per_w = n_full // NW
    n_extra = n_full - k_per_w * NW
    n_pairs = k_per_w // 2
    mesh = plsc.VectorSubcoreMesh(core_axis_name="c", subcore_axis_name="s")

    @functools.partial(
        pl.kernel,
        mesh=mesh,
        out_type=jax.ShapeDtypeStruct((VOC_NUM * VOC_DIM,), jnp.float32),
        scratch_types=[
            pltpu.VMEM((VOC_DIM, C), jnp.float32),
            pltpu.VMEM((VOC_DIM, C), jnp.float32),
            pltpu.VMEM((C * VOC_DIM,), jnp.float32),
            pltpu.VMEM((C * VOC_DIM,), jnp.float32),
            pltpu.VMEM((rem, VOC_DIM), jnp.float32),
            pltpu.VMEM((16 * 16,), jnp.int32),
            pltpu.VMEM((16 * 16,), jnp.int32),
            pltpu.VMEM((16,), jnp.int32),
            pltpu.SemaphoreType.DMA,
            pltpu.SemaphoreType.DMA,
            pltpu.SemaphoreType.DMA,
            pltpu.SemaphoreType.DMA,
        ],
        compiler_params=pltpu.CompilerParams(needs_layout_passes=False),
    )
    def transpose_k(tableT_hbm, rem_hbm, out_hbm, buf0_v, buf1_v,
                    pk0_v, pk1_v, rem_v, gd_v, sd_v, lane_v,
                    gi0, gi1, go0, go1):
        bufs = (buf0_v, buf1_v)
        pks = (pk0_v, pk1_v)
        wid = lax.axis_index("s") * NC + lax.axis_index("c")
        gis = (gi0, gi1)
        gos = (go0, go1)
        lane = lax.iota(jnp.int32, 16)
        lane64 = lane * VOC_DIM
        for s in range(16):
            diag = (lane + s) & 15
            gd_v[pl.ds(s * 16, 16)] = diag
            sd_v[pl.ds(s * 16, 16)] = lane64 + diag
        lane_v[pl.ds(0, 16)] = lane

        def col_off(c):
            return pl.multiple_of(c * C, C)

        def out_off(c):
            return pl.multiple_of(c * (C * VOC_DIM), C * VOC_DIM)

        def fire_in(c, b):
            pltpu.async_copy(
                tableT_hbm.at[:, pl.ds(col_off(c), C)], bufs[b], gis[b])

        def wait_in(c, b):
            pltpu.make_async_copy(
                tableT_hbm.at[:, pl.ds(col_off(c), C)], bufs[b],
                gis[b]).wait()

        def fire_out(c, b):
            pltpu.async_copy(
                pks[b], out_hbm.at[pl.ds(out_off(c), C * VOC_DIM)],
                gos[b])

        def wait_out(c, b):
            pltpu.make_async_copy(
                pks[b], out_hbm.at[pl.ds(out_off(c), C * VOC_DIM)],
                gos[b]).wait()

        def transpose_chunk(b, n_cols=C):
            @plsc.parallel_loop(0, 16 * (VOC_DIM // 16), unroll=2)
            def sd_body(i):
                s = lax.shift_right_logical(i, 2)
                d0 = lax.bitwise_and(i, 3)
                off = pl.multiple_of(s * 16, 16)
                gd = gd_v[pl.ds(off, 16)]
                sd = sd_v[pl.ds(off, 16)]
                lv = lane_v[pl.ds(0, 16)]
                d_base = d0 * 16
                dvec = gd + d_base
                sbase = sd + d_base
                jv = lv
                for j0 in range(n_cols // 16):
                    v = plsc.load_gather(bufs[b], [dvec, jv])
                    plsc.store_scatter(
                        pks[b], [sbase + (j0 * 16 * VOC_DIM)], v)
                    if j0 + 1 < n_cols // 16:
                        jv = jv + 16

        fire_in(wid, 0)

        def pair_body(u, carry):
            c0 = wid + (2 * u) * NW
            c1 = c0 + NW
            c2 = c0 + 2 * NW
            wait_in(c0, 0)
            fire_in(c1, 1)
            pl.when(u > 0)(lambda: wait_out(c0 - 2 * NW, 0))
            transpose_chunk(0)
            fire_out(c0, 0)
            wait_in(c1, 1)
            pl.when(u + 1 < n_pairs)(lambda: fire_in(c2, 0))
            pl.when(u > 0)(lambda: wait_out(c1 - 2 * NW, 1))
            transpose_chunk(1)
            fire_out(c1, 1)
            return carry

        lax.fori_loop(0, n_pairs, pair_body, 0)
        wait_out(wid + (2 * n_pairs - 2) * NW, 0)
        wait_out(wid + (2 * n_pairs - 1) * NW, 1)

        @pl.when(wid < n_extra)
        def _():
            c = k_per_w * NW + wid
            pltpu.sync_copy(
                tableT_hbm.at[:, pl.ds(col_off(c), C)], buf0_v)
            transpose_chunk(0)
            pltpu.sync_copy(
                pk0_v, out_hbm.at[pl.ds(out_off(c), C * VOC_DIM)])

        if rem:
            @pl.when(wid == n_extra)
            def _():
                pltpu.sync_copy(rem_hbm, rem_v)
                for r in range(rem):
                    for q in range(VOC_DIM // 16):
                        pk0_v[pl.ds(r * VOC_DIM + q * 16, 16)] = (
                            rem_v[r, pl.ds(q * 16, 16)]
                        )
                pltpu.sync_copy(
                    pk0_v.at[pl.ds(0, rem * VOC_DIM)],
                    out_hbm.at[pl.ds(n_full * C * VOC_DIM,
                                     rem * VOC_DIM)])

    return transpose_k


def _make_sc_gather(B_total):
    info = plsc.get_sparse_core_info()
    NC, NS = info.num_cores, info.num_subcores
    NW = NC * NS
    b_per_w = B_total // NW
    n_chunks = b_per_w // CHUNK
    mesh = plsc.VectorSubcoreMesh(core_axis_name="c", subcore_axis_name="s")

    @functools.partial(
        pl.kernel,
        mesh=mesh,
        out_type=jax.ShapeDtypeStruct((B_total, VOC_DIM), jnp.float32),
        scratch_types=[
            pltpu.VMEM((b_per_w,), jnp.int32),
            pltpu.VMEM((2, CHUNK, VOC_DIM), jnp.float32),
            pltpu.SemaphoreType.DMA,
            pltpu.SemaphoreType.DMA,
        ],
        compiler_params=pltpu.CompilerParams(use_tc_tiling_on_sc=False),
    )
    def gather_k(idx_hbm, table_hbm, out_hbm, idx_v, rows_v, sem0, sem1):
        wid = lax.axis_index("s") * NC + lax.axis_index("c")
        base = wid * b_per_w
        pltpu.sync_copy(idx_hbm.at[pl.ds(base, b_per_w)], idx_v)
        sems = (sem0, sem1)
        copies = [None, None]
        for j in range(n_chunks + 1):
            if j < n_chunks:
                b = j & 1
                copies[b] = pltpu.async_copy(
                    table_hbm.at[idx_v.at[pl.ds(j * CHUNK, CHUNK)]],
                    rows_v.at[b],
                    sems[b],
                )
            if j >= 1:
                b2 = (j - 1) & 1
                copies[b2].wait()
                pltpu.sync_copy(
                    rows_v.at[b2],
                    out_hbm.at[pl.ds(base + (j - 1) * CHUNK, CHUNK)],
                )

    return gather_k


_sc_transpose = None
_sc_gather = None


def _get_kernels():
    global _sc_transpose, _sc_gather
    if _sc_transpose is None:
        _sc_transpose = _make_sc_transpose()
        _sc_gather = _make_sc_gather(N_LOOKUPS * BATCH)
    return _sc_transpose, _sc_gather


def _proj_body(g_ref, wt_ref, o_ref):
    acc = jnp.dot(g_ref[0], wt_ref[0], preferred_element_type=jnp.float32)
    for k in range(1, N_LOOKUPS):
        acc += jnp.dot(g_ref[k], wt_ref[k], preferred_element_type=jnp.float32)
    o_ref[...] = acc


def _tc_project(gathered2, wt_diag):
    BP = BATCH // 2
    BB = 1024
    grid = (BP // BB,)
    return pl.pallas_call(
        _proj_body,
        grid=grid,
        in_specs=[
            pl.BlockSpec((N_LOOKUPS, BB, 2 * VOC_DIM),
                         lambda i: (0, i, 0)),
            pl.BlockSpec((N_LOOKUPS, 2 * VOC_DIM, 2 * VOC_DIM),
                         lambda i: (0, 0, 0)),
        ],
        out_specs=pl.BlockSpec((BB, 2 * VOC_DIM), lambda i: (i, 0)),
        out_shape=jax.ShapeDtypeStruct((BATCH // 2, 2 * VOC_DIM),
                                       jnp.float32),
    )(gathered2, wt_diag)


def kernel(x1, x2, x4, x5, codebook, W1, W2, W3, W4):
    transpose, gather = _get_kernels()
    idx_all = jnp.concatenate([x1, x2, x4, x5]).astype(jnp.int32)
    n_rem = VOC_NUM % C
    rem_block = lax.slice(codebook, (VOC_NUM - n_rem, 0),
                          (VOC_NUM, VOC_DIM))
    packed = transpose(codebook.T, rem_block)
    table_lin = packed.reshape(VOC_NUM, VOC_DIM)
    gathered = gather(idx_all, table_lin)
    gathered2 = gathered.reshape(N_LOOKUPS, BATCH // 2, 2 * VOC_DIM)
    wt_stack = jnp.stack([W1.T, W2.T, W3.T, W4.T])
    wt_diag = jnp.zeros((N_LOOKUPS, 2 * VOC_DIM, 2 * VOC_DIM),
                        jnp.float32)
    wt_diag = wt_diag.at[:, :VOC_DIM, :VOC_DIM].set(wt_stack)
    wt_diag = wt_diag.at[:, VOC_DIM:, VOC_DIM:].set(wt_stack)
    out2 = _tc_project(gathered2, wt_diag)
    return out2.reshape(BATCH, VOC_DIM)


# speedup vs baseline: 3.6374x; 1.0123x over previous
import functools

import jax
import jax.numpy as jnp
from jax import lax
from jax.experimental import pallas as pl
from jax.experimental.pallas import tpu as pltpu
from jax.experimental.pallas import tpu_sc as plsc

VOC_NUM = 1000000
VOC_DIM = 64
BATCH = 16384
N_LOOKUPS = 4
CHUNK = 128
C = 128


def _make_sc_transpose():
    info = plsc.get_sparse_core_info()
    NC, NS = info.num_cores, info.num_subcores
    NW = NC * NS
    n_full = VOC_NUM // C
    rem = VOC_NUM - n_full * C
    k_per_w = n_full // NW
    n_extra = n_full - k_per_w * NW
    n_pairs = k_per_w // 2
    mesh = plsc.VectorSubcoreMesh(core_axis_name="c", subcore_axis_name="s")

    @functools.partial(
        pl.kernel,
        mesh=mesh,
        out_type=jax.ShapeDtypeStruct((VOC_NUM * VOC_DIM,), jnp.float32),
        scratch_types=[
            pltpu.VMEM((VOC_DIM, C), jnp.float32),
            pltpu.VMEM((VOC_DIM, C), jnp.float32),
            pltpu.VMEM((C * VOC_DIM,), jnp.float32),
            pltpu.VMEM((C * VOC_DIM,), jnp.float32),
            pltpu.VMEM((rem, VOC_DIM), jnp.float32),
            pltpu.VMEM((16 * 16,), jnp.int32),
            pltpu.VMEM((16 * 16,), jnp.int32),
            pltpu.VMEM((16,), jnp.int32),
            pltpu.SemaphoreType.DMA,
            pltpu.SemaphoreType.DMA,
            pltpu.SemaphoreType.DMA,
            pltpu.SemaphoreType.DMA,
        ],
        compiler_params=pltpu.CompilerParams(needs_layout_passes=False),
    )
    def transpose_k(tableT_hbm, rem_hbm, out_hbm, buf0_v, buf1_v,
                    pk0_v, pk1_v, rem_v, gd_v, sd_v, lane_v,
                    gi0, gi1, go0, go1):
        bufs = (buf0_v, buf1_v)
        pks = (pk0_v, pk1_v)
        wid = lax.axis_index("s") * NC + lax.axis_index("c")
        gis = (gi0, gi1)
        gos = (go0, go1)
        lane = lax.iota(jnp.int32, 16)
        lane64 = lane * VOC_DIM
        for s in range(16):
            diag = (lane + s) & 15
            gd_v[pl.ds(s * 16, 16)] = diag
            sd_v[pl.ds(s * 16, 16)] = lane64 + diag
        lane_v[pl.ds(0, 16)] = lane

        def col_off(c):
            return pl.multiple_of(c * C, C)

        def out_off(c):
            return pl.multiple_of(c * (C * VOC_DIM), C * VOC_DIM)

        def fire_in(c, b):
            pltpu.async_copy(
                tableT_hbm.at[:, pl.ds(col_off(c), C)], bufs[b], gis[b])

        def wait_in(c, b):
            pltpu.make_async_copy(
                tableT_hbm.at[:, pl.ds(col_off(c), C)], bufs[b],
                gis[b]).wait()

        def fire_out(c, b):
            pltpu.async_copy(
                pks[b], out_hbm.at[pl.ds(out_off(c), C * VOC_DIM)],
                gos[b])

        def wait_out(c, b):
            pltpu.make_async_copy(
                pks[b], out_hbm.at[pl.ds(out_off(c), C * VOC_DIM)],
                gos[b]).wait()

        def transpose_chunk(b, n_cols=C):
            @plsc.parallel_loop(0, 16 * (VOC_DIM // 16), unroll=2)
            def sd_body(i):
                s = lax.shift_right_logical(i, 2)
                d0 = lax.bitwise_and(i, 3)
                off = pl.multiple_of(s * 16, 16)
                gd = gd_v[pl.ds(off, 16)]
                sd = sd_v[pl.ds(off, 16)]
                lv = lane_v[pl.ds(0, 16)]
                d_base = pl.multiple_of(d0 * 16, 16)
                src = bufs[b].at[pl.ds(d_base, 16)]
                sbase = sd + d0 * 16
                jv = lv
                for j0 in range(n_cols // 16):
                    v = plsc.load_gather(src, [gd, jv])
                    plsc.store_scatter(
                        pks[b], [sbase + (j0 * 16 * VOC_DIM)], v)
                    if j0 + 1 < n_cols // 16:
                        jv = jv + 16

        fire_in(wid, 0)

        def pair_body(u, carry):
            c0 = wid + (2 * u) * NW
            c1 = c0 + NW
            c2 = c0 + 2 * NW
            wait_in(c0, 0)
            fire_in(c1, 1)
            pl.when(u > 0)(lambda: wait_out(c0 - 2 * NW, 0))
            transpose_chunk(0)
            fire_out(c0, 0)
            wait_in(c1, 1)
            pl.when(u + 1 < n_pairs)(lambda: fire_in(c2, 0))
            pl.when(u > 0)(lambda: wait_out(c1 - 2 * NW, 1))
            transpose_chunk(1)
            fire_out(c1, 1)
            return carry

        lax.fori_loop(0, n_pairs, pair_body, 0)
        wait_out(wid + (2 * n_pairs - 2) * NW, 0)
        wait_out(wid + (2 * n_pairs - 1) * NW, 1)

        @pl.when(wid < n_extra)
        def _():
            c = k_per_w * NW + wid
            pltpu.sync_copy(
                tableT_hbm.at[:, pl.ds(col_off(c), C)], buf0_v)
            transpose_chunk(0)
            pltpu.sync_copy(
                pk0_v, out_hbm.at[pl.ds(out_off(c), C * VOC_DIM)])

        if rem:
            @pl.when(wid == n_extra)
            def _():
                pltpu.sync_copy(rem_hbm, rem_v)
                for r in range(rem):
                    for q in range(VOC_DIM // 16):
                        pk0_v[pl.ds(r * VOC_DIM + q * 16, 16)] = (
                            rem_v[r, pl.ds(q * 16, 16)]
                        )
                pltpu.sync_copy(
                    pk0_v.at[pl.ds(0, rem * VOC_DIM)],
                    out_hbm.at[pl.ds(n_full * C * VOC_DIM,
                                     rem * VOC_DIM)])

    return transpose_k


def _make_sc_gather(B_total):
    info = plsc.get_sparse_core_info()
    NC, NS = info.num_cores, info.num_subcores
    NW = NC * NS
    b_per_w = B_total // NW
    n_chunks = b_per_w // CHUNK
    mesh = plsc.VectorSubcoreMesh(core_axis_name="c", subcore_axis_name="s")

    @functools.partial(
        pl.kernel,
        mesh=mesh,
        out_type=jax.ShapeDtypeStruct((B_total, VOC_DIM), jnp.float32),
        scratch_types=[
            pltpu.VMEM((b_per_w,), jnp.int32),
            pltpu.VMEM((2, CHUNK, VOC_DIM), jnp.float32),
            pltpu.SemaphoreType.DMA,
            pltpu.SemaphoreType.DMA,
        ],
        compiler_params=pltpu.CompilerParams(use_tc_tiling_on_sc=False),
    )
    def gather_k(idx_hbm, table_hbm, out_hbm, idx_v, rows_v, sem0, sem1):
        wid = lax.axis_index("s") * NC + lax.axis_index("c")
        base = wid * b_per_w
        pltpu.sync_copy(idx_hbm.at[pl.ds(base, b_per_w)], idx_v)
        sems = (sem0, sem1)
        copies = [None, None]
        for j in range(n_chunks + 1):
            if j < n_chunks:
                b = j & 1
                copies[b] = pltpu.async_copy(
                    table_hbm.at[idx_v.at[pl.ds(j * CHUNK, CHUNK)]],
                    rows_v.at[b],
                    sems[b],
                )
            if j >= 1:
                b2 = (j - 1) & 1
                copies[b2].wait()
                pltpu.sync_copy(
                    rows_v.at[b2],
                    out_hbm.at[pl.ds(base + (j - 1) * CHUNK, CHUNK)],
                )

    return gather_k


_sc_transpose = None
_sc_gather = None


def _get_kernels():
    global _sc_transpose, _sc_gather
    if _sc_transpose is None:
        _sc_transpose = _make_sc_transpose()
        _sc_gather = _make_sc_gather(N_LOOKUPS * BATCH)
    return _sc_transpose, _sc_gather


def _proj_body(g_ref, wt_ref, o_ref):
    acc = jnp.dot(g_ref[0], wt_ref[0], preferred_element_type=jnp.float32)
    for k in range(1, N_LOOKUPS):
        acc += jnp.dot(g_ref[k], wt_ref[k], preferred_element_type=jnp.float32)
    o_ref[...] = acc


def _tc_project(gathered2, wt_diag):
    BP = BATCH // 2
    BB = 1024
    grid = (BP // BB,)
    return pl.pallas_call(
        _proj_body,
        grid=grid,
        in_specs=[
            pl.BlockSpec((N_LOOKUPS, BB, 2 * VOC_DIM),
                         lambda i: (0, i, 0)),
            pl.BlockSpec((N_LOOKUPS, 2 * VOC_DIM, 2 * VOC_DIM),
                         lambda i: (0, 0, 0)),
        ],
        out_specs=pl.BlockSpec((BB, 2 * VOC_DIM), lambda i: (i, 0)),
        out_shape=jax.ShapeDtypeStruct((BATCH // 2, 2 * VOC_DIM),
                                       jnp.float32),
    )(gathered2, wt_diag)


def kernel(x1, x2, x4, x5, codebook, W1, W2, W3, W4):
    transpose, gather = _get_kernels()
    idx_all = jnp.concatenate([x1, x2, x4, x5]).astype(jnp.int32)
    n_rem = VOC_NUM % C
    rem_block = lax.slice(codebook, (VOC_NUM - n_rem, 0),
                          (VOC_NUM, VOC_DIM))
    packed = transpose(codebook.T, rem_block)
    table_lin = packed.reshape(VOC_NUM, VOC_DIM)
    gathered = gather(idx_all, table_lin)
    gathered2 = gathered.reshape(N_LOOKUPS, BATCH // 2, 2 * VOC_DIM)
    wt_stack = jnp.stack([W1.T, W2.T, W3.T, W4.T])
    wt_diag = jnp.zeros((N_LOOKUPS, 2 * VOC_DIM, 2 * VOC_DIM),
                        jnp.float32)
    wt_diag = wt_diag.at[:, :VOC_DIM, :VOC_DIM].set(wt_stack)
    wt_diag = wt_diag.at[:, VOC_DIM:, VOC_DIM:].set(wt_stack)
    out2 = _tc_project(gathered2, wt_diag)
    return out2.reshape(BATCH, VOC_DIM)


# speedup vs baseline: 5.3395x; 1.4679x over previous
import functools

import jax
import jax.numpy as jnp
from jax import lax
from jax.experimental import pallas as pl
from jax.experimental.pallas import tpu as pltpu
from jax.experimental.pallas import tpu_sc as plsc

VOC_NUM = 1000000
VOC_DIM = 64
BATCH = 16384
N_LOOKUPS = 4
CHUNK = 128
C = 128


def _make_sc_transpose():
    info = plsc.get_sparse_core_info()
    NC, NS = info.num_cores, info.num_subcores
    NW = NC * NS
    n_full = VOC_NUM // C
    rem = VOC_NUM - n_full * C
    k_per_w = n_full // NW
    n_extra = n_full - k_per_w * NW
    n_pairs = k_per_w // 2
    mesh = plsc.VectorSubcoreMesh(core_axis_name="c", subcore_axis_name="s")

    @functools.partial(
        pl.kernel,
        mesh=mesh,
        out_type=jax.ShapeDtypeStruct((VOC_NUM * VOC_DIM,), jnp.float32),
        scratch_types=[
            pltpu.VMEM((VOC_DIM, C), jnp.float32),
            pltpu.VMEM((VOC_DIM, C), jnp.float32),
            pltpu.VMEM((VOC_DIM, C), jnp.float32),
            pltpu.VMEM((VOC_DIM, C), jnp.float32),
            pltpu.VMEM((C * VOC_DIM,), jnp.float32),
            pltpu.VMEM((C * VOC_DIM,), jnp.float32),
            pltpu.VMEM((C * VOC_DIM,), jnp.float32),
            pltpu.VMEM((C * VOC_DIM,), jnp.float32),
            pltpu.VMEM((rem, VOC_DIM), jnp.float32),
            pltpu.VMEM((16 * 16,), jnp.int32),
            pltpu.VMEM((16 * 16,), jnp.int32),
            pltpu.VMEM((16,), jnp.int32),
            pltpu.SemaphoreType.DMA,
            pltpu.SemaphoreType.DMA,
            pltpu.SemaphoreType.DMA,
            pltpu.SemaphoreType.DMA,
            pltpu.SemaphoreType.DMA,
            pltpu.SemaphoreType.DMA,
            pltpu.SemaphoreType.DMA,
            pltpu.SemaphoreType.DMA,
        ],
        compiler_params=pltpu.CompilerParams(needs_layout_passes=False),
    )
    def transpose_k(tableT_hbm, rem_hbm, out_hbm, buf0_v, buf1_v,
                    buf2_v, buf3_v, pk0_v, pk1_v, pk2_v, pk3_v,
                    rem_v, gd_v, sd_v, lane_v,
                    gi0, gi1, gi2, gi3, go0, go1, go2, go3):
        bufs = (buf0_v, buf1_v, buf2_v, buf3_v)
        pks = (pk0_v, pk1_v, pk2_v, pk3_v)
        wid = lax.axis_index("s") * NC + lax.axis_index("c")
        gis = (gi0, gi1, gi2, gi3)
        gos = (go0, go1, go2, go3)
        lane = lax.iota(jnp.int32, 16)
        lane64 = lane * VOC_DIM
        for s in range(16):
            diag = (lane + s) & 15
            gd_v[pl.ds(s * 16, 16)] = diag
            sd_v[pl.ds(s * 16, 16)] = lane64 + diag
        lane_v[pl.ds(0, 16)] = lane

        def col_off(c):
            return pl.multiple_of(c * C, C)

        def out_off(c):
            return pl.multiple_of(c * (C * VOC_DIM), C * VOC_DIM)

        def fire_in(c, b):
            pltpu.async_copy(
                tableT_hbm.at[:, pl.ds(col_off(c), C)], bufs[b], gis[b])

        def wait_in(c, b):
            pltpu.make_async_copy(
                tableT_hbm.at[:, pl.ds(col_off(c), C)], bufs[b],
                gis[b]).wait()

        def fire_out(c, b):
            pltpu.async_copy(
                pks[b], out_hbm.at[pl.ds(out_off(c), C * VOC_DIM)],
                gos[b])

        def wait_out(c, b):
            pltpu.make_async_copy(
                pks[b], out_hbm.at[pl.ds(out_off(c), C * VOC_DIM)],
                gos[b]).wait()

        def transpose_chunk(b, n_cols=C):
            @plsc.parallel_loop(0, 16 * (VOC_DIM // 16), unroll=2)
            def sd_body(i):
                s = lax.shift_right_logical(i, 2)
                d0 = lax.bitwise_and(i, 3)
                off = pl.multiple_of(s * 16, 16)
                gd = gd_v[pl.ds(off, 16)]
                sd = sd_v[pl.ds(off, 16)]
                lv = lane_v[pl.ds(0, 16)]
                d_base = pl.multiple_of(d0 * 16, 16)
                src = bufs[b].at[pl.ds(d_base, 16)]
                sbase = sd + d0 * 16
                jv = lv
                for j0 in range(n_cols // 16):
                    v = plsc.load_gather(src, [gd, jv])
                    plsc.store_scatter(
                        pks[b], [sbase + (j0 * 16 * VOC_DIM)], v)
                    if j0 + 1 < n_cols // 16:
                        jv = jv + 16

        n_quads = k_per_w // 4
        for q in range(3):
            fire_in(wid + q * NW, q)

        def quad_body(u, carry):
            for q in range(4):
                k = 4 * u + q
                c = wid + k * NW
                pl.when(k + 3 < k_per_w)(
                    lambda: fire_in(c + 3 * NW, (q + 3) & 3))
                wait_in(c, q)
                pl.when(u > 0)(lambda: wait_out(c - 4 * NW, q))
                transpose_chunk(q)
                fire_out(c, q)
            return carry

        lax.fori_loop(0, n_quads, quad_body, 0)
        for q in range(4):
            wait_out(wid + (4 * (n_quads - 1) + q) * NW, q)

        @pl.when(wid < n_extra)
        def _():
            c = k_per_w * NW + wid
            pltpu.sync_copy(
                tableT_hbm.at[:, pl.ds(col_off(c), C)], buf0_v)
            transpose_chunk(0)
            pltpu.sync_copy(
                pk0_v, out_hbm.at[pl.ds(out_off(c), C * VOC_DIM)])

        if rem:
            @pl.when(wid == n_extra)
            def _():
                pltpu.sync_copy(rem_hbm, rem_v)
                for r in range(rem):
                    for q in range(VOC_DIM // 16):
                        pk0_v[pl.ds(r * VOC_DIM + q * 16, 16)] = (
                            rem_v[r, pl.ds(q * 16, 16)]
                        )
                pltpu.sync_copy(
                    pk0_v.at[pl.ds(0, rem * VOC_DIM)],
                    out_hbm.at[pl.ds(n_full * C * VOC_DIM,
                                     rem * VOC_DIM)])

    return transpose_k


def _make_sc_gather(B_total):
    info = plsc.get_sparse_core_info()
    NC, NS = info.num_cores, info.num_subcores
    NW = NC * NS
    b_per_w = B_total // NW
    n_chunks = b_per_w // CHUNK
    mesh = plsc.VectorSubcoreMesh(core_axis_name="c", subcore_axis_name="s")

    @functools.partial(
        pl.kernel,
        mesh=mesh,
        out_type=jax.ShapeDtypeStruct((B_total, VOC_DIM), jnp.float32),
        scratch_types=[
            pltpu.VMEM((b_per_w,), jnp.int32),
            pltpu.VMEM((2, CHUNK, VOC_DIM), jnp.float32),
            pltpu.SemaphoreType.DMA,
            pltpu.SemaphoreType.DMA,
        ],
        compiler_params=pltpu.CompilerParams(use_tc_tiling_on_sc=False),
    )
    def gather_k(idx_hbm, table_hbm, out_hbm, idx_v, rows_v, sem0, sem1):
        wid = lax.axis_index("s") * NC + lax.axis_index("c")
        base = wid * b_per_w
        pltpu.sync_copy(idx_hbm.at[pl.ds(base, b_per_w)], idx_v)
        sems = (sem0, sem1)
        copies = [None, None]
        for j in range(n_chunks + 1):
            if j < n_chunks:
                b = j & 1
                copies[b] = pltpu.async_copy(
                    table_hbm.at[idx_v.at[pl.ds(j * CHUNK, CHUNK)]],
                    rows_v.at[b],
                    sems[b],
                )
            if j >= 1:
                b2 = (j - 1) & 1
                copies[b2].wait()
                pltpu.sync_copy(
                    rows_v.at[b2],
                    out_hbm.at[pl.ds(base + (j - 1) * CHUNK, CHUNK)],
                )

    return gather_k


_sc_transpose = None
_sc_gather = None


def _get_kernels():
    global _sc_transpose, _sc_gather
    if _sc_transpose is None:
        _sc_transpose = _make_sc_transpose()
        _sc_gather = _make_sc_gather(N_LOOKUPS * BATCH)
    return _sc_transpose, _sc_gather


def _proj_body(g_ref, wt_ref, o_ref):
    acc = jnp.dot(g_ref[0], wt_ref[0], preferred_element_type=jnp.float32)
    for k in range(1, N_LOOKUPS):
        acc += jnp.dot(g_ref[k], wt_ref[k], preferred_element_type=jnp.float32)
    o_ref[...] = acc


def _tc_project(gathered2, wt_diag):
    BP = BATCH // 2
    BB = 1024
    grid = (BP // BB,)
    return pl.pallas_call(
        _proj_body,
        grid=grid,
        in_specs=[
            pl.BlockSpec((N_LOOKUPS, BB, 2 * VOC_DIM),
                         lambda i: (0, i, 0)),
            pl.BlockSpec((N_LOOKUPS, 2 * VOC_DIM, 2 * VOC_DIM),
                         lambda i: (0, 0, 0)),
        ],
        out_specs=pl.BlockSpec((BB, 2 * VOC_DIM), lambda i: (i, 0)),
        out_shape=jax.ShapeDtypeStruct((BATCH // 2, 2 * VOC_DIM),
                                       jnp.float32),
    )(gathered2, wt_diag)


def kernel(x1, x2, x4, x5, codebook, W1, W2, W3, W4):
    transpose, gather = _get_kernels()
    idx_all = jnp.concatenate([x1, x2, x4, x5]).astype(jnp.int32)
    n_rem = VOC_NUM % C
    rem_block = lax.slice(codebook, (VOC_NUM - n_rem, 0),
                          (VOC_NUM, VOC_DIM))
    packed = transpose(codebook.T, rem_block)
    table_lin = packed.reshape(VOC_NUM, VOC_DIM)
    gathered = gather(idx_all, table_lin)
    gathered2 = gathered.reshape(N_LOOKUPS, BATCH // 2, 2 * VOC_DIM)
    wt_stack = jnp.stack([W1.T, W2.T, W3.T, W4.T])
    wt_diag = jnp.zeros((N_LOOKUPS, 2 * VOC_DIM, 2 * VOC_DIM),
                        jnp.float32)
    wt_diag = wt_diag.at[:, :VOC_DIM, :VOC_DIM].set(wt_stack)
    wt_diag = wt_diag.at[:, VOC_DIM:, VOC_DIM:].set(wt_stack)
    out2 = _tc_project(gathered2, wt_diag)
    return out2.reshape(BATCH, VOC_DIM)
